# baseline (speedup 1.0000x reference)
import jax
import jax.numpy as jnp
from jax import lax
from jax.experimental import pallas as pl
from jax.experimental.pallas import tpu as pltpu
import jax.experimental.pallas.tpu_sc as plsc

N = 1000000
L = 16
NW = 16
SB = 256
NSB = 244
CHUNK = SB * NSB
LAST = N - (NW - 1) * CHUNK
BUF = SB * SB
GRP = SB * L


def _lane():
    return lax.iota(jnp.int32, L)


def _splat_sum(v):
    for k in (1, 2, 4, 8):
        v = v + jnp.take_along_axis(v, _lane() ^ k, axis=0)
    return v


def _last_lane(v):
    return jnp.take_along_axis(v, jnp.full((L,), L - 1, jnp.int32), axis=0)


def _tree16(ref, base):
    vs = [ref[pl.ds(base + k * L, L)] for k in range(L)]
    while len(vs) > 1:
        vs = [a + b for a, b in zip(vs[0::2], vs[1::2])]
    return vs[0]


def _body(p_hbm, c_hbm, out_hbm, data_v, c_v, parts_f, parts_i, stage_f,
          stage_i, shared_f, shared_i):
    w = lax.axis_index("s") * 1 + lax.axis_index("c")
    zero = jnp.zeros((L,), jnp.float32)

    def zero_body(i, _):
        data_v[pl.ds(LAST + i * L, L)] = zero
        return 0

    lax.fori_loop(0, (BUF - LAST) // L, zero_body, 0)

    @pl.when(w != NW - 1)
    def _():
        def zb(i, _):
            data_v[pl.ds(CHUNK + i * L, L)] = zero
            return 0
        lax.fori_loop(0, (LAST - CHUNK) // L, zb, 0)

    @pl.when(w == NW - 1)
    def _():
        pltpu.sync_copy(p_hbm.at[pl.ds((NW - 1) * CHUNK + CHUNK, LAST - CHUNK)],
                        data_v.at[pl.ds(CHUNK, LAST - CHUNK)])

    pltpu.sync_copy(p_hbm.at[pl.ds(w * CHUNK, CHUNK)], data_v.at[pl.ds(0, CHUNK)])
    pltpu.sync_copy(c_hbm, c_v)

    def group_body(g, gs):
        lane = _lane()
        acc = zero
        base = g * GRP
        for j in range(L):
            acc = acc + _tree16(data_v, base + j * SB)
        s = jnp.sum(acc)
        return gs + jnp.where(lane == g, s, zero)

    gs = lax.fori_loop(0, L, group_body, zero)
    local_vec = _splat_sum(gs)

    stage_f[...] = local_vec
    pltpu.sync_copy(stage_f, shared_f.at[pl.ds(w * L, L)])
    plsc.subcore_barrier()
    pltpu.sync_copy(shared_f, parts_f)

    total_vec = zero
    prefix_vec = zero
    for i in range(NW):
        row = parts_f[pl.ds(i * L, L)]
        total_vec = total_vec + row
        prefix_vec = prefix_vec + jnp.where(i < w, row, zero)

    r_vec = c_v[...] * total_vec - prefix_vec

    csg = plsc.cumsum(gs)
    mg = csg <= r_vec
    jg_vec = plsc.all_reduce_population_count(mg)
    bg_vec = _splat_sum(jnp.where(mg, gs, zero))
    jg = jnp.minimum(jnp.max(jg_vec), L - 1)

    lane = _lane()
    ms_vec = zero
    mid_base = jg * GRP
    for j in range(L):
        s = jnp.sum(_tree16(data_v, mid_base + j * SB))
        ms_vec = ms_vec + jnp.where(lane == j, s, zero)
    csm = plsc.cumsum(ms_vec) + bg_vec
    mm = csm <= r_vec
    jm_vec = plsc.all_reduce_population_count(mm)
    bd_run_vec = bg_vec + _splat_sum(jnp.where(mm, ms_vec, zero))
    j_vec = jg_vec * L + jm_vec
    limit_vec = jnp.where(w == NW - 1, BUF // SB - 9, NSB)

    base = jnp.minimum(jnp.max(j_vec), BUF // SB - 10) * SB
    fc_vec = bd_run_vec
    fcnt_vec = jnp.zeros((L,), jnp.int32)
    for k in range(SB // L):
        v = data_v[pl.ds(base + k * L, L)]
        cs = plsc.cumsum(v) + fc_vec
        fcnt_vec = fcnt_vec + plsc.all_reduce_population_count(cs <= r_vec)
        fc_vec = _last_lane(cs)

    count_vec = SB * jnp.minimum(j_vec, limit_vec) + jnp.where(
        j_vec < limit_vec, fcnt_vec, 0)

    stage_i[...] = count_vec
    pltpu.sync_copy(stage_i, shared_i.at[pl.ds(w * L, L)])
    plsc.subcore_barrier()
    pltpu.sync_copy(shared_i, parts_i)

    idx_vec = jnp.zeros((L,), jnp.int32)
    for i in range(NW):
        idx_vec = idx_vec + parts_i[pl.ds(i * L, L)]
    idx_vec = jnp.minimum(idx_vec, N - 1)

    @pl.when(w == 0)
    def _():
        stage_i[...] = idx_vec
        pltpu.sync_copy(stage_i, out_hbm)


@jax.jit
def _sc_count(p, c_vec):
    mesh = plsc.VectorSubcoreMesh(
        core_axis_name="c", subcore_axis_name="s", num_cores=1, num_subcores=NW
    )
    f = pl.kernel(
        _body,
        out_type=jax.ShapeDtypeStruct((L,), jnp.int32),
        mesh=mesh,
        compiler_params=pltpu.CompilerParams(needs_layout_passes=False),
        scratch_types=[
            pltpu.VMEM((BUF,), jnp.float32),
            pltpu.VMEM((L,), jnp.float32),
            pltpu.VMEM((NW * L,), jnp.float32),
            pltpu.VMEM((NW * L,), jnp.int32),
            pltpu.VMEM((L,), jnp.float32),
            pltpu.VMEM((L,), jnp.int32),
            pltpu.VMEM_SHARED((NW * L,), jnp.float32),
            pltpu.VMEM_SHARED((NW * L,), jnp.int32),
        ],
    )
    return f(p, c_vec)


def kernel(probabilities):
    c = jax.random.uniform(jax.random.key(42), (), dtype=jnp.float32)
    res = _sc_count(probabilities, jnp.full((L,), c, jnp.float32))
    return res[0]


# speedup vs baseline: 1.4132x; 1.4132x over previous
import jax
import jax.numpy as jnp
from jax import lax
from jax.experimental import pallas as pl
from jax.experimental.pallas import tpu as pltpu
import jax.experimental.pallas.tpu_sc as plsc

N = 1000000
L = 16
NW = 16
SB = 256
NSB = 244
CHUNK = SB * NSB
LAST = N - (NW - 1) * CHUNK
BUF = SB * SB
NPIECE = 4
GP = 4
PIECES = [CHUNK // NPIECE] * 3 + [CHUNK - 3 * (CHUNK // NPIECE)]


def _lane():
    return lax.iota(jnp.int32, L)


def _splat_sum(v):
    for k in (1, 2, 4, 8):
        v = v + jnp.take_along_axis(v, _lane() ^ k, axis=0)
    return v


def _last_lane(v):
    return jnp.take_along_axis(v, jnp.full((L,), L - 1, jnp.int32), axis=0)


def _body(p_hbm, c_hbm, out_hbm, data_v, c_v, sb_v, parts_f, parts_i, stage_f,
          stage_i, shared_f, shared_i):
    w = lax.axis_index("s") * 1 + lax.axis_index("c")
    zero = jnp.zeros((L,), jnp.float32)

    def zero_body(i, _):
        data_v[pl.ds(LAST + i * L, L)] = zero
        return 0

    lax.fori_loop(0, (BUF - LAST) // L, zero_body, 0)

    @pl.when(w != NW - 1)
    def _():
        def zb(i, _):
            data_v[pl.ds(CHUNK + i * L, L)] = zero
            return 0
        lax.fori_loop(0, (LAST - CHUNK) // L, zb, 0)

    @pl.when(w == NW - 1)
    def _():
        pltpu.sync_copy(p_hbm.at[pl.ds((NW - 1) * CHUNK + CHUNK, LAST - CHUNK)],
                        data_v.at[pl.ds(CHUNK, LAST - CHUNK)])

    pltpu.sync_copy(p_hbm.at[pl.ds(w * CHUNK, CHUNK)], data_v.at[pl.ds(0, CHUNK)])
    pltpu.sync_copy(c_hbm, c_v)

    def group_body(g, acc_tot):
        lane = _lane()
        packed = zero
        base = g * (SB * L)
        for j in range(L):
            sb_base = base + j * SB
            vs = [data_v[pl.ds(sb_base + k * L, L)] for k in range(SB // L)]
            while len(vs) > 1:
                vs = [a + b for a, b in zip(vs[0::2], vs[1::2])]
            s = jnp.sum(vs[0])
            packed = packed + jnp.where(lane == j, s, zero)
        sb_v[pl.ds(g * L, L)] = packed
        return acc_tot + packed

    acc_tot = lax.fori_loop(0, L, group_body, zero)
    local_vec = _splat_sum(acc_tot)

    stage_f[...] = local_vec
    pltpu.sync_copy(stage_f, shared_f.at[pl.ds(w * L, L)])
    plsc.subcore_barrier()
    pltpu.sync_copy(shared_f, parts_f)

    total_vec = zero
    prefix_vec = zero
    for i in range(NW):
        row = parts_f[pl.ds(i * L, L)]
        total_vec = total_vec + row
        prefix_vec = prefix_vec + jnp.where(i < w, row, zero)

    r_vec = c_v[...] * total_vec - prefix_vec

    run_vec = zero
    j_vec = jnp.zeros((L,), jnp.int32)
    below_vec = zero
    for t in range(L):
        v = sb_v[pl.ds(t * L, L)]
        cs = plsc.cumsum(v) + run_vec
        m = cs <= r_vec
        j_vec = j_vec + plsc.all_reduce_population_count(m)
        below_vec = below_vec + jnp.where(m, v, zero)
        run_vec = _last_lane(cs)

    bd_run_vec = _splat_sum(below_vec)
    limit_vec = jnp.where(w == NW - 1, BUF // SB - 9, NSB)

    base = jnp.minimum(jnp.max(j_vec), BUF // SB - 10) * SB
    fc_vec = bd_run_vec
    fcnt_vec = jnp.zeros((L,), jnp.int32)
    for k in range(SB // L):
        v = data_v[pl.ds(base + k * L, L)]
        cs = plsc.cumsum(v) + fc_vec
        fcnt_vec = fcnt_vec + plsc.all_reduce_population_count(cs <= r_vec)
        fc_vec = _last_lane(cs)

    count_vec = SB * jnp.minimum(j_vec, limit_vec) + jnp.where(
        j_vec < limit_vec, fcnt_vec, 0)

    stage_i[...] = count_vec
    pltpu.sync_copy(stage_i, shared_i.at[pl.ds(w * L, L)])
    plsc.subcore_barrier()
    pltpu.sync_copy(shared_i, parts_i)

    idx_vec = jnp.zeros((L,), jnp.int32)
    for i in range(NW):
        idx_vec = idx_vec + parts_i[pl.ds(i * L, L)]
    idx_vec = jnp.minimum(idx_vec, N - 1)

    @pl.when(w == 0)
    def _():
        stage_i[...] = idx_vec
        pltpu.sync_copy(stage_i, out_hbm)


@jax.jit
def _sc_count(p, c_vec):
    mesh = plsc.VectorSubcoreMesh(
        core_axis_name="c", subcore_axis_name="s", num_cores=1, num_subcores=NW
    )
    f = pl.kernel(
        _body,
        out_type=jax.ShapeDtypeStruct((L,), jnp.int32),
        mesh=mesh,
        compiler_params=pltpu.CompilerParams(needs_layout_passes=False),
        scratch_types=[
            pltpu.VMEM((BUF,), jnp.float32),
            pltpu.VMEM((L,), jnp.float32),
            pltpu.VMEM((SB,), jnp.float32),
            pltpu.VMEM((NW * L,), jnp.float32),
            pltpu.VMEM((NW * L,), jnp.int32),
            pltpu.VMEM((L,), jnp.float32),
            pltpu.VMEM((L,), jnp.int32),
            pltpu.VMEM_SHARED((NW * L,), jnp.float32),
            pltpu.VMEM_SHARED((NW * L,), jnp.int32),
        ],
    )
    return f(p, c_vec)


def kernel(probabilities):
    c = jax.random.uniform(jax.random.key(42), (), dtype=jnp.float32)
    res = _sc_count(probabilities, jnp.full((L,), c, jnp.float32))
    return res[0]
